# Initial kernel scaffold; baseline (speedup 1.0000x reference)
import jax
import jax.numpy as jnp
from jax.experimental import pallas as pl


def kernel(tokens, tables):
    raise NotImplementedError("write your pallas kernel here")


# speedup vs baseline: 5.6172x; 5.6172x over previous
import jax
import jax.numpy as jnp
from jax import lax
from jax.experimental import pallas as pl
from jax.experimental.pallas import tpu as pltpu
from jax.experimental.pallas import tpu_sc as plsc

_F, _V, _D = 26, 1000, 64
_B, _T = 1024, 50
_TOT = _B * _T * _F
_SUB = 128
_NSUB = _TOT // _SUB
_NW = 32
_SUB_PER_W = _NSUB // _NW
_K = 13
_G = _SUB_PER_W // _K


def _body(idx_hbm, table_hbm, out_hbm, idx_v, rows_v, gsem, ssem):
    wid = lax.axis_index("s") * 2 + lax.axis_index("c")
    w_base = wid * _SUB_PER_W

    @pl.loop(0, _G)
    def _group(g):
        gg = wid * _G + g
        base = w_base + g * _K
        pltpu.sync_copy(idx_hbm.at[gg], idx_v)
        gathers = [
            pltpu.async_copy(table_hbm.at[idx_v.at[j]], rows_v.at[j], gsem)
            for j in range(_K)
        ]
        for d in gathers:
            d.wait()
        scatters = [
            pltpu.async_copy(
                rows_v.at[j], out_hbm.at[pl.ds((base + j) * _SUB, _SUB)], ssem
            )
            for j in range(_K)
        ]
        for d in scatters:
            d.wait()


_gather = pl.kernel(
    _body,
    out_type=jax.ShapeDtypeStruct((_TOT, _D), jnp.float32),
    mesh=plsc.VectorSubcoreMesh(core_axis_name="c", subcore_axis_name="s"),
    scratch_types=[
        pltpu.VMEM((_K, _SUB), jnp.int32),
        pltpu.VMEM((_K, _SUB, _D), jnp.float32),
        pltpu.SemaphoreType.DMA,
        pltpu.SemaphoreType.DMA,
    ],
    compiler_params=pltpu.CompilerParams(use_tc_tiling_on_sc=False),
)


def kernel(tokens, tables):
    f = tables.shape[0]
    flat_idx = tokens + jnp.arange(f, dtype=jnp.int32) * tables.shape[1]
    flat_idx = flat_idx.reshape(_NSUB // _K, _K, _SUB)
    flat_tab = tables.reshape(f * tables.shape[1], tables.shape[2])
    out = _gather(flat_idx, flat_tab)
    return out.reshape(tokens.shape[0], tokens.shape[1], f, tables.shape[2])


# speedup vs baseline: 5.6488x; 1.0056x over previous
import jax
import jax.numpy as jnp
from jax import lax
from jax.experimental import pallas as pl
from jax.experimental.pallas import tpu as pltpu
from jax.experimental.pallas import tpu_sc as plsc

_F, _V, _D = 26, 1000, 64
_B, _T = 1024, 50
_TOT = _B * _T * _F
_SUB = 104
_NSUB = _TOT // _SUB
_NW = 32
_SUB_PER_W = _NSUB // _NW
_K = 8
_GROUP = _K * _SUB
_G = _SUB_PER_W // _K


def _body(idx_hbm, table_hbm, out_hbm, idx_v, rows_v, gsemA, gsemB, ssemA, ssemB):
    wid = lax.axis_index("s") * 2 + lax.axis_index("c")
    w_sub0 = wid * _SUB_PER_W
    w_grp0 = wid * _G

    @pl.loop(0, _G // 2)
    def _pair(h):
        for p, gsem, ssem in ((0, gsemA, ssemA), (1, gsemB, ssemB)):
            g = 2 * h + p

            @pl.when(h > 0)
            def _drain_prev():
                pltpu.make_async_copy(
                    rows_v.at[p], out_hbm.at[pl.ds(0, _GROUP)], ssem
                ).wait()

            pltpu.sync_copy(idx_hbm.at[w_grp0 + g], idx_v.at[p])
            for j in range(_K):
                pltpu.async_copy(
                    table_hbm.at[idx_v.at[p, j]],
                    rows_v.at[p, pl.ds(j * _SUB, _SUB)],
                    gsem,
                )

        for p, gsem, ssem in ((0, gsemA, ssemA), (1, gsemB, ssemB)):
            g = 2 * h + p
            base_row = (w_sub0 + g * _K) * _SUB
            for j in range(_K):
                pltpu.make_async_copy(
                    table_hbm.at[idx_v.at[p, j]],
                    rows_v.at[p, pl.ds(j * _SUB, _SUB)],
                    gsem,
                ).wait()
            pltpu.async_copy(rows_v.at[p], out_hbm.at[pl.ds(base_row, _GROUP)], ssem)

    pltpu.make_async_copy(rows_v.at[0], out_hbm.at[pl.ds(0, _GROUP)], ssemA).wait()
    pltpu.make_async_copy(rows_v.at[1], out_hbm.at[pl.ds(0, _GROUP)], ssemB).wait()


_gather = pl.kernel(
    _body,
    out_type=jax.ShapeDtypeStruct((_TOT, _D), jnp.float32),
    mesh=plsc.VectorSubcoreMesh(core_axis_name="c", subcore_axis_name="s"),
    scratch_types=[
        pltpu.VMEM((2, _K, _SUB), jnp.int32),
        pltpu.VMEM((2, _GROUP, _D), jnp.float32),
        pltpu.SemaphoreType.DMA,
        pltpu.SemaphoreType.DMA,
        pltpu.SemaphoreType.DMA,
        pltpu.SemaphoreType.DMA,
    ],
    compiler_params=pltpu.CompilerParams(use_tc_tiling_on_sc=False),
)


def kernel(tokens, tables):
    f = tables.shape[0]
    flat_idx = tokens + jnp.arange(f, dtype=jnp.int32) * tables.shape[1]
    flat_idx = flat_idx.reshape(_NSUB // _K, _K, _SUB)
    flat_tab = tables.reshape(f * tables.shape[1], tables.shape[2])
    out = _gather(flat_idx, flat_tab)
    return out.reshape(tokens.shape[0], tokens.shape[1], f, tables.shape[2])
